# Initial kernel scaffold; baseline (speedup 1.0000x reference)
import jax
import jax.numpy as jnp
from jax.experimental import pallas as pl


def kernel(in_feat, edge_index, Wl0, b0, Wr0, Wl1, b1, Wr1, Wl2, b2, Wr2):
    raise NotImplementedError("write your pallas kernel here")


# speedup vs baseline: 14.1836x; 14.1836x over previous
import functools

import jax
import jax.numpy as jnp
from jax import lax
from jax.experimental import pallas as pl
from jax.experimental.pallas import tpu as pltpu
from jax.experimental.pallas import tpu_sc as plsc

N = 10000
E = 320000
D = 128

NC = 2
NS = 16
NW = NC * NS

K = 120
CH = 84
E_PAD = NW * CH * K
PAD_ROWS = 112
N_PAD = N + PAD_ROWS
ROWS_PER_TILE = N_PAD // NS
OUT_ROWS_PER_TILE = 624
NBUF = 3
NDST = NBUF + 1


def _sc_body(h_hbm, src_hbm, dst_hbm, zeros_hbm, out_hbm,
             idx_v, rows_v, acc_sh, gsem, ssem, isem):
    c = lax.axis_index("c")
    s = lax.axis_index("s")
    w = c * NS + s

    def fire_idx(chunk, src_slot, dst_slot):
        base = (w * CH + chunk) * K
        pltpu.async_copy(src_hbm.at[pl.ds(base, K)], idx_v.at[src_slot],
                         isem)
        pltpu.async_copy(dst_hbm.at[pl.ds(base, K)], idx_v.at[NBUF + dst_slot],
                         isem)

    def wait_idx(src_slot, dst_slot):
        pltpu.make_async_copy(src_hbm.at[pl.ds(0, K)], idx_v.at[src_slot],
                              isem).wait()
        pltpu.make_async_copy(dst_hbm.at[pl.ds(0, K)],
                              idx_v.at[NBUF + dst_slot], isem).wait()

    def scatter_desc(buf, dst_slot):
        return pltpu.make_async_copy(rows_v.at[buf],
                                     acc_sh.at[idx_v.at[NBUF + dst_slot]],
                                     ssem)

    for t in range(NBUF):
        fire_idx(t, t, t)
    for t in range(NBUF - 1):
        wait_idx(t, t)
        pltpu.async_copy(h_hbm.at[idx_v.at[t]], rows_v.at[t], gsem)
    pltpu.sync_copy(zeros_hbm.at[pl.ds(s * ROWS_PER_TILE, ROWS_PER_TILE)],
                    acc_sh.at[pl.ds(s * ROWS_PER_TILE, ROWS_PER_TILE)])
    plsc.subcore_barrier()

    @pl.loop(0, CH, step=NBUF)
    def _chunks(j0):
        for i in range(NBUF):
            j = j0 + i
            b = i
            @pl.when(j > 0)
            def _():
                scatter_desc((b + 2) % NBUF, (j + NDST - 1) % NDST).wait()
            @pl.when(j + 2 < CH)
            def _():
                wait_idx((b + 2) % NBUF, (j + 2) % NDST)
                pltpu.async_copy(h_hbm.at[idx_v.at[(b + 2) % NBUF]],
                                 rows_v.at[(b + 2) % NBUF], gsem)
            pltpu.make_async_copy(h_hbm.at[idx_v.at[b]], rows_v.at[b],
                                  gsem).wait()
            @pl.when(j + NBUF < CH)
            def _():
                fire_idx(j + NBUF, b, (j + NBUF) % NDST)
            pltpu.async_copy(rows_v.at[b], acc_sh.at[idx_v.at[NBUF + j % NDST]],
                             ssem, add=True)

    scatter_desc((CH - 1) % NBUF, (CH - 1) % NDST).wait()

    plsc.subcore_barrier()
    base = s * OUT_ROWS_PER_TILE
    pltpu.sync_copy(acc_sh.at[pl.ds(base, OUT_ROWS_PER_TILE)],
                    out_hbm.at[pl.ds(c * N + base, OUT_ROWS_PER_TILE)])
    tail = NS * OUT_ROWS_PER_TILE
    @pl.when(s == NS - 1)
    def _():
        pltpu.sync_copy(acc_sh.at[pl.ds(tail, N - tail)],
                        out_hbm.at[pl.ds(c * N + tail, N - tail)])


_sc_aggregate = pl.kernel(
    _sc_body,
    out_type=jax.ShapeDtypeStruct((NC * N, D), jnp.float32),
    mesh=plsc.VectorSubcoreMesh(core_axis_name="c", subcore_axis_name="s"),
    scratch_types=[
        pltpu.VMEM((NBUF + NDST, K), jnp.int32),
        pltpu.VMEM((NBUF, K, D), jnp.float32),
        pltpu.VMEM_SHARED((N_PAD, D), jnp.float32),
        pltpu.SemaphoreType.DMA,
        pltpu.SemaphoreType.DMA,
        pltpu.SemaphoreType.DMA,
    ],
)


R = 1000
_row_spec = pl.BlockSpec((R, D), lambda i: (i, 0))
_full_spec = pl.BlockSpec((D, D), lambda i: (0, 0))


def _tc_combine_body(relu, p0_ref, p1_ref, h_ref, wl_ref, b_ref, wr_ref,
                     o_ref):
    aggr = p0_ref[...] + p1_ref[...]
    out = jnp.dot(aggr, wl_ref[...], preferred_element_type=jnp.float32)
    out = out + jnp.dot(h_ref[...], wr_ref[...],
                        preferred_element_type=jnp.float32)
    out = out + b_ref[...]
    if relu:
        out = jnp.maximum(out, 0.0)
    o_ref[...] = out


def _tc_combine(p, h, Wl, b, Wr, relu):
    p1_spec = pl.BlockSpec((R, D), lambda i: (i + N // R, 0))
    return pl.pallas_call(
        functools.partial(_tc_combine_body, relu),
        grid=(N // R,),
        in_specs=[_row_spec, p1_spec, _row_spec, _full_spec,
                  pl.BlockSpec((1, D), lambda i: (0, 0)), _full_spec],
        out_specs=_row_spec,
        out_shape=jax.ShapeDtypeStruct((N, D), jnp.float32),
    )(p, p, h, Wl, b.reshape(1, D), Wr)


def kernel(in_feat, edge_index, Wl0, b0, Wr0, Wl1, b1, Wr1, Wl2, b2, Wr2):
    src = edge_index[0].astype(jnp.int32)
    dst = edge_index[1].astype(jnp.int32)
    npad = E_PAD - E
    pad_ids = jnp.arange(npad, dtype=jnp.int32)
    src_p = jnp.concatenate([src, (pad_ids * 37) % N])
    dst_p = jnp.concatenate([dst, N + (pad_ids % PAD_ROWS)])
    zeros = jnp.zeros((N_PAD, D), jnp.float32)

    h = in_feat
    for Wl, b, Wr, relu in ((Wl0, b0, Wr0, True), (Wl1, b1, Wr1, True),
                            (Wl2, b2, Wr2, False)):
        p = _sc_aggregate(h, src_p, dst_p, zeros)
        h = _tc_combine(p, h, Wl, b, Wr, relu)
    return h
